# baseline (speedup 1.0000x reference)
import functools

import jax
import jax.numpy as jnp
from jax import lax
from jax.experimental import pallas as pl
from jax.experimental.pallas import tpu as pltpu
from jax.experimental.pallas import tpu_sc as plsc

NUM_CORES = 2
NUM_WORKERS = 32

T_DIM = 200
S_DIM = 4096
S_BLK = 128
C_DIM = S_DIM // S_BLK
D = 64
UNITS = T_DIM * C_DIM
U_PER_W = UNITS // NUM_WORKERS


@jax.jit
def _embed(x_t_flat, table):
    mesh = plsc.VectorSubcoreMesh(core_axis_name="c", subcore_axis_name="s")

    @functools.partial(
        pl.kernel,
        mesh=mesh,
        out_type=jax.ShapeDtypeStruct((T_DIM, 8, C_DIM, 8, S_BLK), jnp.float32),
        compiler_params=pltpu.CompilerParams(
            use_tc_tiling_on_sc=False, needs_layout_passes=False
        ),
        scratch_types=[
            *[pltpu.VMEM((S_BLK,), jnp.int32) for _ in range(2)],
            *[pltpu.VMEM((S_BLK, D), jnp.float32) for _ in range(2)],
            *[pltpu.VMEM((8, 8, S_BLK), jnp.float32) for _ in range(2)],
            *[pltpu.SemaphoreType.DMA for _ in range(4)],
        ],
    )
    def k(x_hbm, table_hbm, out_hbm, idx0, idx1, rows0, rows1, ob0, ob1,
          g0, g1, s0, s1):
        idxs, rows, obs = (idx0, idx1), (rows0, rows1), (ob0, ob1)
        gsem, ssem = (g0, g1), (s0, s1)
        wid = lax.axis_index("s") * NUM_CORES + lax.axis_index("c")
        u_base = wid * U_PER_W
        iota = lax.iota(jnp.int32, 16)

        def load_idx(u, b):
            pltpu.sync_copy(x_hbm.at[pl.ds(u * S_BLK, S_BLK)], idxs[b])

        def fire_gather(b):
            pltpu.async_copy(table_hbm.at[idxs[b]], rows[b], gsem[b])

        def wait_gather(b):
            pltpu.make_async_copy(table_hbm.at[idxs[b]], rows[b], gsem[b]).wait()

        def fire_store(u, b):
            t = u // C_DIM
            c = lax.rem(u, C_DIM)
            pltpu.async_copy(obs[b], out_hbm.at[t, :, c], ssem[b])

        def wait_store(b):
            pltpu.make_async_copy(obs[b], out_hbm.at[0, :, 0], ssem[b]).wait()

        def transpose(b):
            def tr_body(sl0, carry):
                ridx = iota + sl0 * 16
                for g in range(8):
                    for ds in range(8):
                        cidx = jnp.full((16,), 8 * g + ds, jnp.int32)
                        v = plsc.load_gather(rows[b], [ridx, cidx])
                        obs[b][g, ds, pl.ds(sl0 * 16, 16)] = v
                return carry

            lax.fori_loop(0, S_BLK // 16, tr_body, 0)

        for b in range(2):
            load_idx(u_base + b, b)
            fire_gather(b)

        def body(i, carry):
            for b in range(2):
                j = 2 * i + b
                u = u_base + j
                wait_gather(b)

                @pl.when(i > 0)
                def _():
                    wait_store(b)

                transpose(b)
                fire_store(u, b)

                @pl.when(j + 2 < U_PER_W)
                def _():
                    load_idx(u + 2, b)
                    fire_gather(b)

            return carry

        lax.fori_loop(0, U_PER_W // 2, body, 0)
        for b in range(2):
            wait_store(b)

    return k(x_t_flat, table)


def kernel(x, table):
    xf = x.T.reshape(-1)
    out5 = _embed(xf, table)
    return out5.transpose(2, 4, 0, 1, 3).reshape(S_DIM, T_DIM, D)


# speedup vs baseline: 1.2102x; 1.2102x over previous
import functools

import jax
import jax.numpy as jnp
from jax import lax
from jax.experimental import pallas as pl
from jax.experimental.pallas import tpu as pltpu
from jax.experimental.pallas import tpu_sc as plsc

NUM_CORES = 2
NUM_WORKERS = 32

T_DIM = 200
S_DIM = 4096
S_BLK = 128
C_DIM = S_DIM // S_BLK
D = 64
UNITS = T_DIM * C_DIM
U_PER_W = UNITS // NUM_WORKERS


@jax.jit
def _embed(x_t_flat, table):
    mesh = plsc.VectorSubcoreMesh(core_axis_name="c", subcore_axis_name="s")

    @functools.partial(
        pl.kernel,
        mesh=mesh,
        out_type=jax.ShapeDtypeStruct((T_DIM, 8, C_DIM, 8, S_BLK), jnp.float32),
        compiler_params=pltpu.CompilerParams(
            use_tc_tiling_on_sc=False, needs_layout_passes=False
        ),
        scratch_types=[
            *[pltpu.VMEM((S_BLK,), jnp.int32) for _ in range(2)],
            *[pltpu.VMEM((S_BLK, D), jnp.float32) for _ in range(2)],
            *[pltpu.VMEM((8, 8, S_BLK), jnp.float32) for _ in range(2)],
            *[pltpu.SemaphoreType.DMA for _ in range(4)],
        ],
    )
    def k(x_hbm, table_hbm, out_hbm, idx0, idx1, rows0, rows1, ob0, ob1,
          g0, g1, s0, s1):
        idxs, rows, obs = (idx0, idx1), (rows0, rows1), (ob0, ob1)
        gsem, ssem = (g0, g1), (s0, s1)
        wid = lax.axis_index("s") * NUM_CORES + lax.axis_index("c")
        u_base = wid * U_PER_W
        iota = lax.iota(jnp.int32, 16)

        def load_idx(u, b):
            pltpu.sync_copy(x_hbm.at[pl.ds(u * S_BLK, S_BLK)], idxs[b])

        def fire_gather(b):
            pltpu.async_copy(table_hbm.at[idxs[b]], rows[b], gsem[b])

        def wait_gather(b):
            pltpu.make_async_copy(table_hbm.at[idxs[b]], rows[b], gsem[b]).wait()

        def fire_store(u, b):
            t = u // C_DIM
            c = lax.rem(u, C_DIM)
            pltpu.async_copy(obs[b], out_hbm.at[t, :, c], ssem[b])

        def wait_store(b):
            pltpu.make_async_copy(obs[b], out_hbm.at[0, :, 0], ssem[b]).wait()

        def transpose(b):
            @plsc.parallel_loop(0, D * (S_BLK // 16), unroll=16)
            def _tr(i):
                d = i >> 3
                sl0 = i & 7
                ridx = iota + sl0 * 16
                cidx = jnp.full((16,), 1, jnp.int32) * d
                v = plsc.load_gather(rows[b], [ridx, cidx])
                obs[b][d >> 3, d & 7, pl.ds(sl0 * 16, 16)] = v

        for b in range(2):
            load_idx(u_base + b, b)
            fire_gather(b)

        def body(i, carry):
            for b in range(2):
                j = 2 * i + b
                u = u_base + j
                wait_gather(b)

                @pl.when(i > 0)
                def _():
                    wait_store(b)

                transpose(b)
                fire_store(u, b)

                @pl.when(j + 2 < U_PER_W)
                def _():
                    load_idx(u + 2, b)
                    fire_gather(b)

            return carry

        lax.fori_loop(0, U_PER_W // 2, body, 0)
        for b in range(2):
            wait_store(b)

    return k(x_t_flat, table)


def kernel(x, table):
    xf = x.T.reshape(-1)
    out5 = _embed(xf, table)
    return out5.transpose(2, 4, 0, 1, 3).reshape(S_DIM, T_DIM, D)


# speedup vs baseline: 1.2401x; 1.0247x over previous
import functools

import jax
import jax.numpy as jnp
from jax import lax
from jax.experimental import pallas as pl
from jax.experimental.pallas import tpu as pltpu
from jax.experimental.pallas import tpu_sc as plsc

NUM_CORES = 2
NUM_WORKERS = 32

T_DIM = 200
S_DIM = 4096
S_BLK = 128
C_DIM = S_DIM // S_BLK
D = 64
UNITS = T_DIM * C_DIM
U_PER_W = UNITS // NUM_WORKERS


@jax.jit
def _embed(x_t_flat, table):
    mesh = plsc.VectorSubcoreMesh(core_axis_name="c", subcore_axis_name="s")

    @functools.partial(
        pl.kernel,
        mesh=mesh,
        out_type=jax.ShapeDtypeStruct((T_DIM, 8, C_DIM, 8, S_BLK), jnp.float32),
        compiler_params=pltpu.CompilerParams(
            use_tc_tiling_on_sc=False, needs_layout_passes=False
        ),
        scratch_types=[
            *[pltpu.VMEM((S_BLK,), jnp.int32) for _ in range(2)],
            *[pltpu.VMEM((S_BLK, D), jnp.float32) for _ in range(2)],
            *[pltpu.VMEM((8, 8, S_BLK), jnp.float32) for _ in range(2)],
            *[pltpu.SemaphoreType.DMA for _ in range(4)],
        ],
    )
    def k(x_hbm, table_hbm, out_hbm, idx0, idx1, rows0, rows1, ob0, ob1,
          g0, g1, s0, s1):
        idxs, rows, obs = (idx0, idx1), (rows0, rows1), (ob0, ob1)
        gsem, ssem = (g0, g1), (s0, s1)
        wid = lax.axis_index("s") * NUM_CORES + lax.axis_index("c")
        u_base = wid * U_PER_W
        iota = lax.iota(jnp.int32, 16)

        def load_idx(u, b):
            pltpu.sync_copy(x_hbm.at[pl.ds(u * S_BLK, S_BLK)], idxs[b])

        def fire_gather(b):
            pltpu.async_copy(table_hbm.at[idxs[b]], rows[b], gsem[b])

        def wait_gather(b):
            pltpu.make_async_copy(table_hbm.at[idxs[b]], rows[b], gsem[b]).wait()

        def fire_store(u, b):
            t = u // C_DIM
            c = lax.rem(u, C_DIM)
            pltpu.async_copy(obs[b], out_hbm.at[t, :, c], ssem[b])

        def wait_store(b):
            pltpu.make_async_copy(obs[b], out_hbm.at[0, :, 0], ssem[b]).wait()

        cidxs = [jnp.full((16,), d, jnp.int32) for d in range(D)]

        def transpose(b):
            @plsc.parallel_loop(0, S_BLK // 16, unroll=2)
            def _tr(sl0):
                ridx = iota + sl0 * 16
                off = sl0 * 16
                for g in range(8):
                    for ds in range(8):
                        v = plsc.load_gather(rows[b], [ridx, cidxs[8 * g + ds]])
                        obs[b][g, ds, pl.ds(off, 16)] = v

        for b in range(2):
            load_idx(u_base + b, b)
            fire_gather(b)

        def body(i, carry):
            for b in range(2):
                j = 2 * i + b
                u = u_base + j
                wait_gather(b)

                @pl.when(i > 0)
                def _():
                    wait_store(b)

                transpose(b)
                fire_store(u, b)

                @pl.when(j + 2 < U_PER_W)
                def _():
                    load_idx(u + 2, b)
                    fire_gather(b)

            return carry

        lax.fori_loop(0, U_PER_W // 2, body, 0)
        for b in range(2):
            wait_store(b)

    return k(x_t_flat, table)


def kernel(x, table):
    xf = x.T.reshape(-1)
    out5 = _embed(xf, table)
    return out5.transpose(2, 4, 0, 1, 3).reshape(S_DIM, T_DIM, D)


# speedup vs baseline: 2.2764x; 1.8356x over previous
import functools

import jax
import jax.numpy as jnp
from jax import lax
from jax.experimental import pallas as pl
from jax.experimental.pallas import tpu as pltpu
from jax.experimental.pallas import tpu_sc as plsc

NUM_CORES = 2
NUM_WORKERS = 32

T_DIM = 200
S_DIM = 4096
S_BLK = 128
C_DIM = S_DIM // S_BLK
D = 64
UNITS = T_DIM * C_DIM
U_PER_W = UNITS // NUM_WORKERS


@jax.jit
def _embed(x_t_flat, table):
    mesh = plsc.VectorSubcoreMesh(core_axis_name="c", subcore_axis_name="s")

    @functools.partial(
        pl.kernel,
        mesh=mesh,
        out_type=jax.ShapeDtypeStruct((T_DIM, 8, C_DIM, 8, S_BLK), jnp.float32),
        compiler_params=pltpu.CompilerParams(
            use_tc_tiling_on_sc=False, needs_layout_passes=False
        ),
        scratch_types=[
            *[pltpu.VMEM((S_BLK,), jnp.int32) for _ in range(2)],
            *[pltpu.VMEM((S_BLK, D), jnp.float32) for _ in range(2)],
            *[pltpu.VMEM((8, 8, S_BLK + 1), jnp.float32) for _ in range(2)],
            *[pltpu.SemaphoreType.DMA for _ in range(4)],
        ],
    )
    def k(x_hbm, table_hbm, out_hbm, idx0, idx1, rows0, rows1, ob0, ob1,
          g0, g1, s0, s1):
        idxs, rows, obs = (idx0, idx1), (rows0, rows1), (ob0, ob1)
        gsem, ssem = (g0, g1), (s0, s1)
        wid = lax.axis_index("s") * NUM_CORES + lax.axis_index("c")
        u_base = wid * U_PER_W
        iota = lax.iota(jnp.int32, 16)

        def load_idx(u, b):
            pltpu.sync_copy(x_hbm.at[pl.ds(u * S_BLK, S_BLK)], idxs[b])

        def fire_gather(b):
            pltpu.async_copy(table_hbm.at[idxs[b]], rows[b], gsem[b])

        def wait_gather(b):
            pltpu.make_async_copy(table_hbm.at[idxs[b]], rows[b], gsem[b]).wait()

        def fire_store(u, b):
            t = u // C_DIM
            c = lax.rem(u, C_DIM)
            pltpu.async_copy(
                obs[b].at[:, :, pl.ds(0, S_BLK)], out_hbm.at[t, :, c], ssem[b]
            )

        def wait_store(b):
            pltpu.make_async_copy(
                obs[b].at[:, :, pl.ds(0, S_BLK)], out_hbm.at[0, :, 0], ssem[b]
            ).wait()

        gconst = [(iota + d0) >> 3 for d0 in range(0, D, 16)]
        dsconst = [(iota + d0) & 7 for d0 in range(0, D, 16)]

        def transpose(b):
            @plsc.parallel_loop(0, S_BLK, unroll=4)
            def _tr(sl):
                slv = jnp.full((16,), sl, jnp.int32)
                for q in range(D // 16):
                    v = rows[b][sl, pl.ds(q * 16, 16)]
                    plsc.store_scatter(obs[b], [gconst[q], dsconst[q], slv], v)

        for b in range(2):
            load_idx(u_base + b, b)
            fire_gather(b)

        def body(i, carry):
            for b in range(2):
                j = 2 * i + b
                u = u_base + j
                wait_gather(b)

                @pl.when(i > 0)
                def _():
                    wait_store(b)

                transpose(b)
                fire_store(u, b)

                @pl.when(j + 2 < U_PER_W)
                def _():
                    load_idx(u + 2, b)
                    fire_gather(b)

            return carry

        lax.fori_loop(0, U_PER_W // 2, body, 0)
        for b in range(2):
            wait_store(b)

    return k(x_t_flat, table)


def kernel(x, table):
    xf = x.T.reshape(-1)
    out5 = _embed(xf, table)
    return out5.transpose(2, 4, 0, 1, 3).reshape(S_DIM, T_DIM, D)
